# baseline (speedup 1.0000x reference)
import dataclasses
import functools

import jax
import jax.numpy as jnp
from jax import lax
from jax.experimental import pallas as pl
from jax.experimental.pallas import tpu as pltpu
from jax.experimental.pallas import tpu_sc as plsc

D0, D1 = 2048, 2048
N = D0 * D1
K_KEEP = N // 2
RB = 256
NB = D0 // RB

NTILES = 16
CHUNK = N // NTILES
SUB = 2 * D1
NSUB = CHUNK // SUB
NV = D1 // 16
UNROLL = 8
ROWS = SUB // D1

LEVEL_BINS = (2048, 2048, 512)


def _norm_body(x_ref, norm_ref):
    xx = x_ref[...]
    norm_ref[...] = jnp.sqrt(jnp.sum(xx * xx, axis=0, keepdims=True))


def _suffix_select(cnt_v, nrows, k_needed):
    def step(j, carry):
        cnt_ge, s_hi, tot = carry
        jj = nrows - 1 - j
        v = cnt_v[jj, :]
        suf = lax.rev(lax.cumsum(lax.rev(v, (0,)), axis=0), (0,)) + tot
        ge = suf >= k_needed
        cnt_ge = cnt_ge + jnp.sum(ge.astype(jnp.int32))
        s_hi = s_hi + jnp.sum(jnp.where(ge, jnp.int32(0), v))
        tot = tot + jnp.sum(v)
        return cnt_ge, s_hi, tot

    z = jnp.int32(0)
    cnt_ge, s_hi, _ = lax.fori_loop(0, nrows, step, (z, z, z))
    return cnt_ge - 1, k_needed - s_hi


def _make_sc_select():
    mesh = plsc.VectorSubcoreMesh(core_axis_name="c", subcore_axis_name="s",
                                  num_cores=1)
    scratch = [
        pltpu.VMEM((D1,), jnp.float32),
        pltpu.VMEM((SUB,), jnp.float32),
        pltpu.VMEM((SUB,), jnp.float32),
        pltpu.VMEM((16 * 2048,), jnp.int32),
        pltpu.VMEM((128, 16), jnp.int32),
        pltpu.VMEM((128, 16), jnp.int32),
        pltpu.VMEM((16,), jnp.int32),
        pltpu.VMEM((16,), jnp.int32),
        pltpu.VMEM_SHARED((NTILES, 128, 16), jnp.int32),
        pltpu.SemaphoreType.DMA,
        pltpu.SemaphoreType.DMA,
    ]
    out_type = jax.ShapeDtypeStruct((16,), jnp.int32)

    def body(w_hbm, norm_hbm, t_hbm, norm_v, buf0, buf1, hist_v, merged_v,
             tmp_v, pk_v, tvec_v, slots, sem0, sem1):
        sid = lax.axis_index("s")
        base = sid * CHUNK
        lane_iota = lax.iota(jnp.int32, 16)

        pltpu.sync_copy(norm_hbm, norm_v)

        ps = []
        k_needed = jnp.int32(K_KEEP)
        for level in range(3):
            nbins = LEVEL_BINS[level]
            nrows = nbins // 16
            lane_base = lax.iota(jnp.int32, 16) * nbins
            ones = jnp.ones((16,), jnp.int32)
            if level == 1:
                prefix = ps[0]
            elif level == 2:
                prefix = ps[0] * 2048 + ps[1]

            @pl.loop(0, nbins, step=8)
            def _(i):
                for u in range(8):
                    hist_v[pl.ds((i + u) * 16, 16)] = jnp.zeros((16,),
                                                                jnp.int32)

            def process(buf):
                @pl.loop(0, NV, step=UNROLL)
                def _(c0):
                    offs = [(c0 + u) * 16 for u in range(UNROLL)]
                    nvs = [norm_v[pl.ds(off, 16)] for off in offs]
                    wvs = [buf[pl.ds(r * D1 + off, 16)]
                           for off in offs for r in range(ROWS)]
                    bs = [lax.bitcast_convert_type(
                              jnp.abs(wv) * nvs[i // ROWS], jnp.int32)
                          for i, wv in enumerate(wvs)]
                    if level == 0:
                        idxs = [lax.shift_right_logical(b, 20) + lane_base
                                for b in bs]
                        for idx in idxs:
                            plsc.addupdate_scatter(hist_v, [idx], ones)
                    elif level == 1:
                        matches = [lax.shift_right_logical(b, 20) == prefix
                                   for b in bs]
                        idxs = [jnp.bitwise_and(
                                    lax.shift_right_logical(b, 9), 2047)
                                + lane_base for b in bs]
                        for idx, match in zip(idxs, matches):
                            plsc.addupdate_scatter(hist_v, [idx], ones,
                                                   mask=match)
                    else:
                        matches = [lax.shift_right_logical(b, 9) == prefix
                                   for b in bs]
                        idxs = [jnp.bitwise_and(b, 511) + lane_base
                                for b in bs]
                        for idx, match in zip(idxs, matches):
                            plsc.addupdate_scatter(hist_v, [idx], ones,
                                                   mask=match)

            pltpu.async_copy(w_hbm.at[pl.ds(base, SUB)], buf0, sem0)
            pltpu.async_copy(w_hbm.at[pl.ds(base + SUB, SUB)], buf1, sem1)

            @pl.loop(0, NSUB // 2)
            def _(s):
                for bi in range(2):
                    buf = (buf0, buf1)[bi]
                    sem = (sem0, sem1)[bi]
                    cur = s * 2 + bi
                    pltpu.make_async_copy(
                        w_hbm.at[pl.ds(base + cur * SUB, SUB)], buf,
                        sem).wait()
                    process(buf)

                    @pl.when(cur + 2 < NSUB)
                    def _():
                        pltpu.async_copy(
                            w_hbm.at[pl.ds(base + (cur + 2) * SUB, SUB)],
                            buf, sem)

            @pl.loop(0, nrows, step=2)
            def _(j0):
                for j in (j0, j0 + 1):
                    vals = [hist_v[pl.ds(l * nbins + j * 16, 16)]
                            for l in range(16)]
                    while len(vals) > 1:
                        vals = [vals[i] + vals[i + 1]
                                for i in range(0, len(vals), 2)]
                    merged_v[j, :] = vals[0]

            pltpu.sync_copy(merged_v.at[pl.ds(0, nrows)],
                            slots.at[sid, pl.ds(0, nrows)])
            plsc.subcore_barrier()

            @pl.when(sid == 0)
            def _():
                for t in range(1, NTILES):
                    pltpu.sync_copy(slots.at[t, pl.ds(0, nrows)],
                                    tmp_v.at[pl.ds(0, nrows)])

                    @pl.loop(0, nrows, step=2)
                    def _(j0):
                        for j in (j0, j0 + 1):
                            merged_v[j, :] = merged_v[j, :] + tmp_v[j, :]

            p, k_needed = _suffix_select(merged_v, nrows, k_needed)

            @pl.when(sid == 0)
            def _():
                pk_v[...] = jnp.where(lane_iota == 0, p, 0)
                pltpu.sync_copy(pk_v, slots.at[0, 0])

            plsc.subcore_barrier()
            pltpu.sync_copy(slots.at[0, 0], pk_v)
            p = jnp.sum(jnp.where(lane_iota == 0, pk_v[...], 0))
            ps.append(p)

        t = ps[0] * (1 << 20) + ps[1] * (1 << 9) + ps[2]
        tvec_v[...] = jnp.full((16,), jnp.int32(0)) + t

        @pl.when(sid == 0)
        def _():
            pltpu.sync_copy(tvec_v, t_hbm)

    cp = pltpu.CompilerParams()
    if "needs_layout_passes" in pltpu.CompilerParams.__dataclass_fields__:
        cp = dataclasses.replace(cp, needs_layout_passes=False)
    return pl.kernel(body, out_type=out_type, mesh=mesh, scratch_types=scratch,
                     compiler_params=cp)


_sc_select = _make_sc_select()


def _mask_body(w_ref, norm_ref, t_ref, out_ref):
    t = t_ref[0, 0]
    bits = lax.bitcast_convert_type(jnp.abs(w_ref[...]) * norm_ref[...],
                                    jnp.int32)
    out_ref[...] = jnp.where(bits >= t, w_ref[...], 0.0)


@jax.jit
def kernel(x, weight):
    norm2d = pl.pallas_call(
        _norm_body,
        out_shape=jax.ShapeDtypeStruct((1, D1), jnp.float32),
    )(x)
    tvec = _sc_select(weight.reshape(N), norm2d.reshape(D1))
    out = pl.pallas_call(
        _mask_body,
        grid=(NB,),
        in_specs=[
            pl.BlockSpec((RB, D1), lambda i: (i, 0)),
            pl.BlockSpec((1, D1), lambda i: (0, 0)),
            pl.BlockSpec(memory_space=pltpu.SMEM),
        ],
        out_specs=pl.BlockSpec((RB, D1), lambda i: (i, 0)),
        out_shape=jax.ShapeDtypeStruct((D0, D1), jnp.float32),
    )(weight, norm2d, tvec.reshape(1, 16))
    return out


# speedup vs baseline: 1.1845x; 1.1845x over previous
import dataclasses
import functools

import jax
import jax.numpy as jnp
from jax import lax
from jax.experimental import pallas as pl
from jax.experimental.pallas import tpu as pltpu
from jax.experimental.pallas import tpu_sc as plsc

D0, D1 = 2048, 2048
N = D0 * D1
K_KEEP = N // 2
RB = 256
NB = D0 // RB

NC, NS, LANES = 2, 16, 16
NW = NC * NS
CHUNK = N // NW
SUB = 2 * D1
NSUB = CHUNK // SUB
NV = D1 // LANES

L2_BINS = 2048
L3_BINS = 512


def _norm_body(x_ref, norm_ref):
    i = pl.program_id(0)

    @pl.when(i == 0)
    def _():
        norm_ref[...] = jnp.zeros_like(norm_ref)

    xx = x_ref[...]
    norm_ref[...] += jnp.sum(xx * xx, axis=0, keepdims=True)

    @pl.when(i == NB - 1)
    def _():
        norm_ref[...] = jnp.sqrt(norm_ref[...])


def _suffix_select(cnt_ref, nbv, k_needed):
    def step(j, carry):
        cnt_ge, s_hi, tot = carry
        jj = nbv - 1 - j
        v = cnt_ref[pl.ds(jj * 16, 16)]
        suf = lax.rev(lax.cumsum(lax.rev(v, (0,)), axis=0), (0,)) + tot
        ge = suf >= k_needed
        cnt_ge = cnt_ge + jnp.sum(ge.astype(jnp.int32))
        s_hi = s_hi + jnp.sum(jnp.where(ge, jnp.int32(0), v))
        tot = tot + jnp.sum(v)
        return cnt_ge, s_hi, tot

    z = jnp.int32(0)
    cnt_ge, s_hi, _ = lax.fori_loop(0, nbv, step, (z, z, z))
    return cnt_ge - 1, k_needed - s_hi


def _make_sc_pass(level):
    nbins = L3_BINS if level == 3 else L2_BINS
    nbv = nbins // 16
    scratch = [
        pltpu.VMEM((D1,), jnp.float32),
        pltpu.VMEM((SUB,), jnp.float32),
        pltpu.VMEM((SUB,), jnp.float32),
        pltpu.VMEM((16 * nbins,), jnp.int32),
        pltpu.VMEM((nbins,), jnp.int32),
        pltpu.SemaphoreType.DMA,
        pltpu.SemaphoreType.DMA,
    ]
    if level >= 2:
        scratch += [
            pltpu.VMEM((NW * L2_BINS,), jnp.int32),
            pltpu.VMEM((L2_BINS,), jnp.int32),
        ]
    mesh = plsc.VectorSubcoreMesh(core_axis_name="c", subcore_axis_name="s")
    out_type = jax.ShapeDtypeStruct((NC, NS, nbins), jnp.int32)

    def body(*refs):
        if level == 1:
            (w_hbm, norm_hbm, out_hbm,
             norm_v, buf0, buf1, hist_v, merged_v, sem0, sem1) = refs
            h_hbms = ()
        elif level == 2:
            (w_hbm, norm_hbm, h1_hbm, out_hbm,
             norm_v, buf0, buf1, hist_v, merged_v, sem0, sem1,
             hp_v, cnt_v) = refs
            h_hbms = (h1_hbm,)
        else:
            (w_hbm, norm_hbm, h1_hbm, h2_hbm, out_hbm,
             norm_v, buf0, buf1, hist_v, merged_v, sem0, sem1,
             hp_v, cnt_v) = refs
            h_hbms = (h1_hbm, h2_hbm)

        cid = lax.axis_index("c")
        sid = lax.axis_index("s")
        base = (cid * NS + sid) * CHUNK

        pltpu.sync_copy(norm_hbm, norm_v)

        ps = []
        k_needed = jnp.int32(K_KEEP)
        for h_hbm in h_hbms:
            pltpu.sync_copy(h_hbm, hp_v)

            @pl.loop(0, L2_BINS // 16)
            def _(j):
                vals = [hp_v[pl.ds(t * L2_BINS + j * 16, 16)]
                        for t in range(NW)]
                while len(vals) > 1:
                    vals = [vals[i] + vals[i + 1]
                            for i in range(0, len(vals), 2)]
                cnt_v[pl.ds(j * 16, 16)] = vals[0]

            p, k_needed = _suffix_select(cnt_v, L2_BINS // 16, k_needed)
            ps.append(p)
        if level == 2:
            prefix = ps[0]
        elif level == 3:
            prefix = ps[0] * 2048 + ps[1]

        @pl.loop(0, nbv * 16, step=8)
        def _(i):
            for u in range(8):
                hist_v[pl.ds((i + u) * 16, 16)] = jnp.zeros((16,), jnp.int32)

        lane_base = lax.iota(jnp.int32, 16) * nbins
        ones = jnp.ones((16,), jnp.int32)

        UNROLL = 8
        ROWS = SUB // D1

        def process(buf):
            @pl.loop(0, NV, step=UNROLL)
            def _(c0):
                offs = [(c0 + u) * 16 for u in range(UNROLL)]
                nvs = [norm_v[pl.ds(off, 16)] for off in offs]
                wvs = [buf[pl.ds(r * D1 + off, 16)]
                       for off in offs for r in range(ROWS)]
                bs = [lax.bitcast_convert_type(jnp.abs(wv) * nvs[i // ROWS],
                                               jnp.int32)
                      for i, wv in enumerate(wvs)]
                if level == 1:
                    idxs = [lax.shift_right_logical(b, 20) + lane_base
                            for b in bs]
                    for idx in idxs:
                        plsc.addupdate_scatter(hist_v, [idx], ones)
                elif level == 2:
                    matches = [lax.shift_right_logical(b, 20) == prefix
                               for b in bs]
                    idxs = [jnp.bitwise_and(lax.shift_right_logical(b, 9),
                                            L2_BINS - 1) + lane_base
                            for b in bs]
                    for idx, match in zip(idxs, matches):
                        plsc.addupdate_scatter(hist_v, [idx], ones, mask=match)
                else:
                    matches = [lax.shift_right_logical(b, 9) == prefix
                               for b in bs]
                    idxs = [jnp.bitwise_and(b, L3_BINS - 1) + lane_base
                            for b in bs]
                    for idx, match in zip(idxs, matches):
                        plsc.addupdate_scatter(hist_v, [idx], ones, mask=match)

        pltpu.async_copy(w_hbm.at[pl.ds(base, SUB)], buf0, sem0)
        pltpu.async_copy(w_hbm.at[pl.ds(base + SUB, SUB)], buf1, sem1)

        @pl.loop(0, NSUB // 2)
        def _(s):
            for bi in range(2):
                buf = (buf0, buf1)[bi]
                sem = (sem0, sem1)[bi]
                cur = s * 2 + bi
                pltpu.make_async_copy(
                    w_hbm.at[pl.ds(base + cur * SUB, SUB)], buf, sem).wait()
                process(buf)

                @pl.when(cur + 2 < NSUB)
                def _():
                    pltpu.async_copy(
                        w_hbm.at[pl.ds(base + (cur + 2) * SUB, SUB)], buf, sem)

        @pl.loop(0, nbv, step=2)
        def _(j0):
            for j in (j0, j0 + 1):
                vals = [hist_v[pl.ds(l * nbins + j * 16, 16)]
                        for l in range(16)]
                while len(vals) > 1:
                    vals = [vals[i] + vals[i + 1]
                            for i in range(0, len(vals), 2)]
                merged_v[pl.ds(j * 16, 16)] = vals[0]

        pltpu.sync_copy(merged_v, out_hbm.at[cid, sid])

    cp = pltpu.CompilerParams()
    if "needs_layout_passes" in pltpu.CompilerParams.__dataclass_fields__:
        cp = dataclasses.replace(cp, needs_layout_passes=False)
    return pl.kernel(body, out_type=out_type, mesh=mesh, scratch_types=scratch,
                     compiler_params=cp)


_sc_pass1 = _make_sc_pass(1)
_sc_pass2 = _make_sc_pass(2)
_sc_pass3 = _make_sc_pass(3)


def _tc_level(cnt, nbins, nbits, k_needed):
    iota = lax.broadcasted_iota(jnp.int32, (1, nbins), 1)

    def suffix_at(b):
        return jnp.sum(jnp.where(iota >= b, cnt, 0))

    def st(_, lohi):
        lo, hi = lohi
        mid = hi - (hi - lo) // 2
        feas = suffix_at(mid) >= k_needed
        return (jnp.where(feas, mid, lo), jnp.where(feas, hi, mid - 1))

    p, _ = lax.fori_loop(0, nbits, st, (jnp.int32(0), jnp.int32(nbins - 1)))
    at_p = jnp.sum(jnp.where(iota == p, cnt, 0))
    return p, k_needed - (suffix_at(p) - at_p)


def _mask_body(w_ref, norm_ref, h1_ref, h2_ref, h3_ref, out_ref):
    cnt1 = jnp.sum(h1_ref[...], axis=0, keepdims=True)
    cnt2 = jnp.sum(h2_ref[...], axis=0, keepdims=True)
    cnt3 = jnp.sum(h3_ref[...], axis=0, keepdims=True)
    p1, k2 = _tc_level(cnt1, L2_BINS, 11, jnp.int32(K_KEEP))
    p2, k3 = _tc_level(cnt2, L2_BINS, 11, k2)
    p3, _ = _tc_level(cnt3, L3_BINS, 9, k3)
    t = p1 * (1 << 20) + p2 * (1 << 9) + p3
    bits = lax.bitcast_convert_type(jnp.abs(w_ref[...]) * norm_ref[...],
                                    jnp.int32)
    out_ref[...] = jnp.where(bits >= t, w_ref[...], 0.0)


@jax.jit
def kernel(x, weight):
    norm2d = pl.pallas_call(
        _norm_body,
        grid=(NB,),
        in_specs=[pl.BlockSpec((RB, D1), lambda i: (i, 0))],
        out_specs=pl.BlockSpec((1, D1), lambda i: (0, 0)),
        out_shape=jax.ShapeDtypeStruct((1, D1), jnp.float32),
    )(x)
    norm1d = norm2d.reshape(D1)
    wf = weight.reshape(N)
    h1 = _sc_pass1(wf, norm1d)
    h2 = _sc_pass2(wf, norm1d, h1.reshape(NW * L2_BINS))
    h3 = _sc_pass3(wf, norm1d, h1.reshape(NW * L2_BINS), h2.reshape(NW * L2_BINS))
    out = pl.pallas_call(
        _mask_body,
        grid=(NB,),
        in_specs=[
            pl.BlockSpec((RB, D1), lambda i: (i, 0)),
            pl.BlockSpec((1, D1), lambda i: (0, 0)),
            pl.BlockSpec((NW, L2_BINS), lambda i: (0, 0)),
            pl.BlockSpec((NW, L2_BINS), lambda i: (0, 0)),
            pl.BlockSpec((NW, L3_BINS), lambda i: (0, 0)),
        ],
        out_specs=pl.BlockSpec((RB, D1), lambda i: (i, 0)),
        out_shape=jax.ShapeDtypeStruct((D0, D1), jnp.float32),
    )(weight, norm2d, h1.reshape(NW, L2_BINS), h2.reshape(NW, L2_BINS),
      h3.reshape(NW, L3_BINS))
    return out


# speedup vs baseline: 1.3412x; 1.1323x over previous
import dataclasses
import functools

import jax
import jax.numpy as jnp
from jax import lax
from jax.experimental import pallas as pl
from jax.experimental.pallas import tpu as pltpu
from jax.experimental.pallas import tpu_sc as plsc

D0, D1 = 2048, 2048
N = D0 * D1
K_KEEP = N // 2
RB = 256
NB = D0 // RB

NC, NS, LANES = 2, 16, 16
NW = NC * NS
CHUNK = N // NW
SUB = 2 * D1
NSUB = CHUNK // SUB
NV = D1 // LANES

L2_BINS = 2048
L3_BINS = 512


def _norm_body(x_ref, norm_ref):
    i = pl.program_id(0)

    @pl.when(i == 0)
    def _():
        norm_ref[...] = jnp.zeros_like(norm_ref)

    xx = x_ref[...]
    norm_ref[...] += jnp.sum(xx * xx, axis=0, keepdims=True)

    @pl.when(i == NB - 1)
    def _():
        norm_ref[...] = jnp.sqrt(norm_ref[...])


def _suffix_select(cnt_ref, nbv, k_needed):
    def step(j, carry):
        cnt_ge, s_hi, tot = carry
        jj = nbv - 1 - j
        v = cnt_ref[pl.ds(jj * 16, 16)]
        suf = lax.rev(lax.cumsum(lax.rev(v, (0,)), axis=0), (0,)) + tot
        ge = suf >= k_needed
        cnt_ge = cnt_ge + jnp.sum(ge.astype(jnp.int32))
        s_hi = s_hi + jnp.sum(jnp.where(ge, jnp.int32(0), v))
        tot = tot + jnp.sum(v)
        return cnt_ge, s_hi, tot

    z = jnp.int32(0)
    cnt_ge, s_hi, _ = lax.fori_loop(0, nbv, step, (z, z, z))
    return cnt_ge - 1, k_needed - s_hi


def _make_sc_pass(level):
    nbins = L3_BINS if level == 3 else L2_BINS
    nbv = nbins // 16
    scratch = [
        pltpu.VMEM((D1,), jnp.float32),
        pltpu.VMEM((SUB,), jnp.float32),
        pltpu.VMEM((SUB,), jnp.float32),
        pltpu.VMEM((16 * nbins,), jnp.int32),
        pltpu.VMEM((nbins,), jnp.int32),
        pltpu.SemaphoreType.DMA,
        pltpu.SemaphoreType.DMA,
    ]
    if level >= 2:
        scratch += [
            pltpu.VMEM((NW * L2_BINS,), jnp.int32),
            pltpu.VMEM((L2_BINS,), jnp.int32),
        ]
    mesh = plsc.VectorSubcoreMesh(core_axis_name="c", subcore_axis_name="s")
    out_type = jax.ShapeDtypeStruct((NC, NS, nbins), jnp.int32)

    def body(*refs):
        if level == 1:
            (w_hbm, norm_hbm, out_hbm,
             norm_v, buf0, buf1, hist_v, merged_v, sem0, sem1) = refs
            h_hbms = ()
        elif level == 2:
            (w_hbm, norm_hbm, h1_hbm, out_hbm,
             norm_v, buf0, buf1, hist_v, merged_v, sem0, sem1,
             hp_v, cnt_v) = refs
            h_hbms = (h1_hbm,)
        else:
            (w_hbm, norm_hbm, h1_hbm, h2_hbm, out_hbm,
             norm_v, buf0, buf1, hist_v, merged_v, sem0, sem1,
             hp_v, cnt_v) = refs
            h_hbms = (h1_hbm, h2_hbm)

        cid = lax.axis_index("c")
        sid = lax.axis_index("s")
        base = (cid * NS + sid) * CHUNK

        pltpu.sync_copy(norm_hbm, norm_v)

        ps = []
        k_needed = jnp.int32(K_KEEP)
        for h_hbm in h_hbms:
            pltpu.sync_copy(h_hbm, hp_v)

            @pl.loop(0, L2_BINS // 16)
            def _(j):
                vals = [hp_v[pl.ds(t * L2_BINS + j * 16, 16)]
                        for t in range(NW)]
                while len(vals) > 1:
                    vals = [vals[i] + vals[i + 1]
                            for i in range(0, len(vals), 2)]
                cnt_v[pl.ds(j * 16, 16)] = vals[0]

            p, k_needed = _suffix_select(cnt_v, L2_BINS // 16, k_needed)
            ps.append(p)
        if level == 2:
            prefix = ps[0]
        elif level == 3:
            prefix = ps[0] * 2048 + ps[1]

        @pl.loop(0, nbv * 16, step=8)
        def _(i):
            for u in range(8):
                hist_v[pl.ds((i + u) * 16, 16)] = jnp.zeros((16,), jnp.int32)

        lane_base = lax.iota(jnp.int32, 16) * nbins
        ones = jnp.ones((16,), jnp.int32)

        UNROLL = 8
        ROWS = SUB // D1

        def process(buf):
            @pl.loop(0, NV, step=UNROLL)
            def _(c0):
                offs = [(c0 + u) * 16 for u in range(UNROLL)]
                nvs = [norm_v[pl.ds(off, 16)] for off in offs]
                wvs = [buf[pl.ds(r * D1 + off, 16)]
                       for off in offs for r in range(ROWS)]
                bs = [lax.bitcast_convert_type(jnp.abs(wv) * nvs[i // ROWS],
                                               jnp.int32)
                      for i, wv in enumerate(wvs)]
                if level == 1:
                    idxs = [lax.shift_right_logical(b, 20) + lane_base
                            for b in bs]
                    for idx in idxs:
                        plsc.addupdate_scatter(hist_v, [idx], ones)
                elif level == 2:
                    matches = [lax.shift_right_logical(b, 20) == prefix
                               for b in bs]
                    idxs = [jnp.bitwise_and(lax.shift_right_logical(b, 9),
                                            L2_BINS - 1) + lane_base
                            for b in bs]
                    for idx, match in zip(idxs, matches):
                        plsc.addupdate_scatter(hist_v, [idx], ones, mask=match)
                else:
                    matches = [lax.shift_right_logical(b, 9) == prefix
                               for b in bs]
                    idxs = [jnp.bitwise_and(b, L3_BINS - 1) + lane_base
                            for b in bs]
                    for idx, match in zip(idxs, matches):
                        plsc.addupdate_scatter(hist_v, [idx], ones, mask=match)

        pltpu.async_copy(w_hbm.at[pl.ds(base, SUB)], buf0, sem0)
        pltpu.async_copy(w_hbm.at[pl.ds(base + SUB, SUB)], buf1, sem1)

        @pl.loop(0, NSUB // 2)
        def _(s):
            for bi in range(2):
                buf = (buf0, buf1)[bi]
                sem = (sem0, sem1)[bi]
                cur = s * 2 + bi
                pltpu.make_async_copy(
                    w_hbm.at[pl.ds(base + cur * SUB, SUB)], buf, sem).wait()
                process(buf)

                @pl.when(cur + 2 < NSUB)
                def _():
                    pltpu.async_copy(
                        w_hbm.at[pl.ds(base + (cur + 2) * SUB, SUB)], buf, sem)

        @pl.loop(0, nbv, step=2)
        def _(j0):
            for j in (j0, j0 + 1):
                vals = [hist_v[pl.ds(l * nbins + j * 16, 16)]
                        for l in range(16)]
                while len(vals) > 1:
                    vals = [vals[i] + vals[i + 1]
                            for i in range(0, len(vals), 2)]
                merged_v[pl.ds(j * 16, 16)] = vals[0]

        pltpu.sync_copy(merged_v, out_hbm.at[cid, sid])

    cp = pltpu.CompilerParams()
    if "needs_layout_passes" in pltpu.CompilerParams.__dataclass_fields__:
        cp = dataclasses.replace(cp, needs_layout_passes=False)
    return pl.kernel(body, out_type=out_type, mesh=mesh, scratch_types=scratch,
                     compiler_params=cp)


_sc_pass1 = _make_sc_pass(1)
_sc_pass2 = _make_sc_pass(2)
_sc_pass3 = _make_sc_pass(3)


def _tc_level(cnt, nbins, nbits, k_needed):
    iota = lax.broadcasted_iota(jnp.int32, (1, nbins), 1)

    def suffix_at(b):
        return jnp.sum(jnp.where(iota >= b, cnt, 0))

    def st(_, lohi):
        lo, hi = lohi
        mid = hi - (hi - lo) // 2
        feas = suffix_at(mid) >= k_needed
        return (jnp.where(feas, mid, lo), jnp.where(feas, hi, mid - 1))

    p, _ = lax.fori_loop(0, nbits, st, (jnp.int32(0), jnp.int32(nbins - 1)))
    at_p = jnp.sum(jnp.where(iota == p, cnt, 0))
    return p, k_needed - (suffix_at(p) - at_p)


def _mask_body(w_ref, norm_ref, h1_ref, h2_ref, h3_ref, out_ref, t_ref):
    @pl.when(pl.program_id(0) == 0)
    def _():
        cnt1 = jnp.sum(h1_ref[...], axis=0, keepdims=True)
        cnt2 = jnp.sum(h2_ref[...], axis=0, keepdims=True)
        cnt3 = jnp.sum(h3_ref[...], axis=0, keepdims=True)
        p1, k2 = _tc_level(cnt1, L2_BINS, 11, jnp.int32(K_KEEP))
        p2, k3 = _tc_level(cnt2, L2_BINS, 11, k2)
        p3, _ = _tc_level(cnt3, L3_BINS, 9, k3)
        t_ref[0] = p1 * (1 << 20) + p2 * (1 << 9) + p3

    t = t_ref[0]
    bits = lax.bitcast_convert_type(jnp.abs(w_ref[...]) * norm_ref[...],
                                    jnp.int32)
    out_ref[...] = jnp.where(bits >= t, w_ref[...], 0.0)


@jax.jit
def kernel(x, weight):
    norm2d = pl.pallas_call(
        _norm_body,
        grid=(NB,),
        in_specs=[pl.BlockSpec((RB, D1), lambda i: (i, 0))],
        out_specs=pl.BlockSpec((1, D1), lambda i: (0, 0)),
        out_shape=jax.ShapeDtypeStruct((1, D1), jnp.float32),
    )(x)
    norm1d = norm2d.reshape(D1)
    wf = weight.reshape(N)
    h1 = _sc_pass1(wf, norm1d)
    h2 = _sc_pass2(wf, norm1d, h1.reshape(NW * L2_BINS))
    h3 = _sc_pass3(wf, norm1d, h1.reshape(NW * L2_BINS), h2.reshape(NW * L2_BINS))
    out = pl.pallas_call(
        _mask_body,
        grid=(NB,),
        in_specs=[
            pl.BlockSpec((RB, D1), lambda i: (i, 0)),
            pl.BlockSpec((1, D1), lambda i: (0, 0)),
            pl.BlockSpec((NW, L2_BINS), lambda i: (0, 0)),
            pl.BlockSpec((NW, L2_BINS), lambda i: (0, 0)),
            pl.BlockSpec((NW, L3_BINS), lambda i: (0, 0)),
        ],
        out_specs=pl.BlockSpec((RB, D1), lambda i: (i, 0)),
        out_shape=jax.ShapeDtypeStruct((D0, D1), jnp.float32),
        scratch_shapes=[pltpu.SMEM((1,), jnp.int32)],
    )(weight, norm2d, h1.reshape(NW, L2_BINS), h2.reshape(NW, L2_BINS),
      h3.reshape(NW, L3_BINS))
    return out
